# Initial kernel scaffold; baseline (speedup 1.0000x reference)
import jax
import jax.numpy as jnp
from jax.experimental import pallas as pl


def kernel(s_emb, t_emb, idx):
    raise NotImplementedError("write your pallas kernel here")


# speedup vs baseline: 8.1382x; 8.1382x over previous
import jax
import jax.numpy as jnp
from jax import lax
from jax.experimental import pallas as pl
from jax.experimental.pallas import tpu as pltpu

_N = 1024
_K = 10
_H = 5


def _fused_kernel(sn_ref, tn_ref, ssc_ref, ssr_ref, ttc_ref, ttr_ref,
                  idxc_ref, idxr_ref, loss_ref,
                  wp, work, wnn, gh, ds, mbuf, wchat):
    f32 = jnp.float32
    iota_row = lax.broadcasted_iota(jnp.int32, (_N, _N), 1)
    iota_col = lax.broadcasted_iota(jnp.int32, (_N, _N), 0)
    eye = iota_col == iota_row

    tn = tn_ref[...]
    gt = lax.dot_general(tn, tn, (((1,), (1,)), ((), ())),
                         preferred_element_type=f32)
    d2t = ttc_ref[...] + ttr_ref[...] - 2.0 * gt
    d2t = jnp.maximum(d2t, 0.0)
    tdist = jnp.where(d2t > 0, jnp.sqrt(jnp.where(d2t > 0, d2t, 1.0)), 0.0)
    wp[...] = jnp.exp(-(tdist * tdist))
    same = idxc_ref[...] == idxr_ref[...]
    work[...] = jnp.where(same, 1.0, wp[...])

    sn = sn_ref[...]
    gs = lax.dot_general(sn, sn, (((1,), (1,)), ((), ())),
                         preferred_element_type=f32)
    d2s = ssc_ref[...] + ssr_ref[...] - 2.0 * gs
    d2s = jnp.maximum(d2s, 0.0)
    ds[...] = jnp.where(d2s > 0, jnp.sqrt(jnp.where(d2s > 0, d2s, 1.0)), 0.0)

    wnn[...] = jnp.zeros((_N, _N), f32)
    gh[...] = jnp.zeros((_N, _N), f32)

    def body(k, carry):
        w = work[...]
        m = jnp.max(w, axis=1, keepdims=True)
        cand = jnp.where(w == m, iota_row, _N)
        j = jnp.min(cand, axis=1, keepdims=True)
        oh = iota_row == j
        work[...] = jnp.where(oh, -jnp.inf, w)
        wnn[...] = jnp.where(oh, 1.0, wnn[...])

        @pl.when(k < _H)
        def _():
            gh[...] = jnp.where(oh, 1.0 / _H, gh[...])

        return carry

    lax.fori_loop(0, _K, body, 0)

    wnn_t = wnn[...].T
    v = jnp.where((wnn[...] > 0) & (wnn_t > 0), 1.0, 0.0)
    denom = jnp.sum(v, axis=1, keepdims=True)
    m_mat = lax.dot_general(v, v, (((1,), (1,)), ((), ())),
                            preferred_element_type=f32)
    mbuf[...] = v * m_mat / jnp.where(denom > 0, denom, 1.0)

    wchat[...] = lax.dot_general(gh[...], mbuf[...], (((1,), (0,)), ((), ())),
                                 preferred_element_type=f32)
    wchat_t = wchat[...].T

    d = ds[...]
    mu = jnp.mean(d, axis=1, keepdims=True)
    s = d / mu
    a = s * s
    r = jnp.maximum(1.0 - s, 0.0)
    b = r * r
    wc = 0.5 * (wchat[...] + wchat_t)
    w_full = 0.5 * (wp[...] + wc)
    term = a * w_full + b * (1.0 - w_full)
    term = jnp.where(eye, 0.0, term)
    total = jnp.sum(jnp.sum(term, axis=1, keepdims=True), axis=0, keepdims=True)
    loss_ref[...] = total / (_N * (_N - 1))


def _run(sn, tn, ssc, ssr, ttc, ttr, idxc, idxr, interpret=False):
    scr = [pltpu.VMEM((_N, _N), jnp.float32)] * 7
    return pl.pallas_call(
        _fused_kernel,
        out_shape=jax.ShapeDtypeStruct((1, 1), jnp.float32),
        scratch_shapes=scr,
        interpret=interpret,
    )(sn, tn, ssc, ssr, ttc, ttr, idxc, idxr)


def kernel(s_emb, t_emb, idx):
    def _norm(x):
        n = jnp.sqrt(jnp.sum(x * x, axis=1, keepdims=True))
        return x / jnp.maximum(n, 1e-12)

    sn = _norm(s_emb)
    tn = _norm(t_emb)
    ss = jnp.sum(sn * sn, axis=1)
    tt = jnp.sum(tn * tn, axis=1)
    idx32 = idx.astype(jnp.int32)
    out = _run(sn, tn,
               ss[:, None], ss[None, :],
               tt[:, None], tt[None, :],
               idx32[:, None], idx32[None, :])
    return out[0, 0]


# speedup vs baseline: 9.0103x; 1.1072x over previous
import jax
import jax.numpy as jnp
from jax import lax
from jax.experimental import pallas as pl
from jax.experimental.pallas import tpu as pltpu

_N = 1024
_K = 10
_H = 5


def _fused_kernel(sn_ref, tn_ref, ssc_ref, ssr_ref, ttc_ref, ttr_ref,
                  idxc_ref, idxr_ref, loss_ref,
                  wp, work, wnn, ds, mbuf, wchat):
    f32 = jnp.float32
    bf16 = jnp.bfloat16
    iota_row = lax.broadcasted_iota(jnp.int32, (_N, _N), 1)
    iota_col = lax.broadcasted_iota(jnp.int32, (_N, _N), 0)
    eye = iota_col == iota_row

    tn = tn_ref[...]
    gt = lax.dot_general(tn, tn, (((1,), (1,)), ((), ())),
                         preferred_element_type=f32)
    d2t = ttc_ref[...] + ttr_ref[...] - 2.0 * gt
    wp[...] = jnp.exp(-jnp.maximum(d2t, 0.0))
    same = idxc_ref[...] == idxr_ref[...]
    work[...] = jnp.where(same, 1.0, wp[...])

    sn = sn_ref[...]
    gs = lax.dot_general(sn, sn, (((1,), (1,)), ((), ())),
                         preferred_element_type=f32)
    d2s = ssc_ref[...] + ssr_ref[...] - 2.0 * gs
    d2s = jnp.maximum(d2s, 0.0)
    ds[...] = jnp.where(d2s > 0, jnp.sqrt(jnp.where(d2s > 0, d2s, 1.0)), 0.0)

    js = []
    for k in range(_K):
        w = work[...]
        m = jnp.max(w, axis=1, keepdims=True)
        j = jnp.min(jnp.where(w == m, iota_row, _N), axis=1, keepdims=True)
        js.append(j)
        if k + 1 < _K:
            work[...] = jnp.where(iota_row == j, -jnp.inf, w)

    hit_half = (iota_row == js[0])
    for k in range(1, _H):
        hit_half = hit_half | (iota_row == js[k])
    hit_rest = (iota_row == js[_H])
    for k in range(_H + 1, _K):
        hit_rest = hit_rest | (iota_row == js[k])
    wnn[...] = jnp.where(hit_half | hit_rest, 1.0, 0.0)
    gh_bf = jnp.where(hit_half, 1.0 / _H, 0.0).astype(bf16)

    wnn_t = wnn[...].T
    v = jnp.where((wnn[...] > 0) & (wnn_t > 0), 1.0, 0.0)
    denom = jnp.sum(v, axis=1, keepdims=True)
    inv_denom = 1.0 / jnp.where(denom > 0, denom, 1.0)
    v_bf = v.astype(bf16)
    m_mat = lax.dot_general(v_bf, v_bf, (((1,), (1,)), ((), ())),
                            preferred_element_type=f32)
    mbuf[...] = v * m_mat * inv_denom

    wchat[...] = lax.dot_general(gh_bf, mbuf[...].astype(bf16),
                                 (((1,), (0,)), ((), ())),
                                 preferred_element_type=f32)
    wchat_t = wchat[...].T

    d = ds[...]
    inv_mu = float(_N) / jnp.sum(d, axis=1, keepdims=True)
    s = d * inv_mu
    a = s * s
    r = jnp.maximum(1.0 - s, 0.0)
    b = r * r
    wc = 0.5 * (wchat[...] + wchat_t)
    w_full = 0.5 * (wp[...] + wc)
    term = b + (a - b) * w_full
    term = jnp.where(eye, 0.0, term)
    total = jnp.sum(jnp.sum(term, axis=1, keepdims=True), axis=0, keepdims=True)
    loss_ref[...] = total / (_N * (_N - 1))


def _run(sn, tn, ssc, ssr, ttc, ttr, idxc, idxr, interpret=False):
    scr = [pltpu.VMEM((_N, _N), jnp.float32)] * 6
    return pl.pallas_call(
        _fused_kernel,
        out_shape=jax.ShapeDtypeStruct((1, 1), jnp.float32),
        scratch_shapes=scr,
        interpret=interpret,
    )(sn, tn, ssc, ssr, ttc, ttr, idxc, idxr)


def kernel(s_emb, t_emb, idx):
    def _norm(x):
        n = jnp.sqrt(jnp.sum(x * x, axis=1, keepdims=True))
        return x / jnp.maximum(n, 1e-12)

    sn = _norm(s_emb)
    tn = _norm(t_emb)
    ss = jnp.sum(sn * sn, axis=1)
    tt = jnp.sum(tn * tn, axis=1)
    idx32 = idx.astype(jnp.int32)
    out = _run(sn, tn,
               ss[:, None], ss[None, :],
               tt[:, None], tt[None, :],
               idx32[:, None], idx32[None, :])
    return out[0, 0]


# speedup vs baseline: 11.4683x; 1.2728x over previous
import jax
import jax.numpy as jnp
from jax import lax
from jax.experimental import pallas as pl
from jax.experimental.pallas import tpu as pltpu

_N = 1024
_K = 10
_H = 5


def _fused_kernel(sn_ref, tn_ref, ssc_ref, ssr_ref, ttc_ref, ttr_ref,
                  idxc_ref, idxr_ref, loss_ref,
                  wp, keys, wnn, ds, mbuf, wchat):
    f32 = jnp.float32
    bf16 = jnp.bfloat16
    i32 = jnp.int32
    iota_row = lax.broadcasted_iota(i32, (_N, _N), 1)
    iota_col = lax.broadcasted_iota(i32, (_N, _N), 0)
    eye = iota_col == iota_row

    tn = tn_ref[...]
    gt = lax.dot_general(tn, tn, (((1,), (1,)), ((), ())),
                         preferred_element_type=f32)
    d2t = ttc_ref[...] + ttr_ref[...] - 2.0 * gt
    wpv = jnp.exp(-jnp.maximum(d2t, 0.0))
    wp[...] = wpv
    same = idxc_ref[...] == idxr_ref[...]

    bits = lax.bitcast_convert_type(jnp.where(same, 1.0, wpv), i32)
    keys[...] = (bits & i32(~1023)) | (i32(1023) - iota_row)

    sn = sn_ref[...]
    gs = lax.dot_general(sn, sn, (((1,), (1,)), ((), ())),
                         preferred_element_type=f32)
    d2s = ssc_ref[...] + ssr_ref[...] - 2.0 * gs
    ds[...] = jnp.sqrt(jnp.maximum(d2s, 0.0))

    int_min = i32(-2147483648)
    m = jnp.max(keys[...], axis=1, keepdims=True)
    m_half = m
    for k in range(1, _K):
        kv = keys[...]
        m = jnp.max(jnp.where(kv < m, kv, int_min), axis=1, keepdims=True)
        if k == _H - 1:
            m_half = m

    kv = keys[...]
    wnn[...] = jnp.where(kv >= m, 1.0, 0.0)
    gh_bf = jnp.where(kv >= m_half, 1.0 / _H, 0.0).astype(bf16)

    wnn_t = wnn[...].T
    v = jnp.where((wnn[...] > 0) & (wnn_t > 0), 1.0, 0.0)
    denom = jnp.sum(v, axis=1, keepdims=True)
    inv_denom = 1.0 / jnp.where(denom > 0, denom, 1.0)
    v_bf = v.astype(bf16)
    m_mat = lax.dot_general(v_bf, v_bf, (((1,), (1,)), ((), ())),
                            preferred_element_type=f32)
    mbuf[...] = v * m_mat * inv_denom

    wchat[...] = lax.dot_general(gh_bf, mbuf[...].astype(bf16),
                                 (((1,), (0,)), ((), ())),
                                 preferred_element_type=f32)
    wchat_t = wchat[...].T

    d = ds[...]
    inv_mu = float(_N) / jnp.sum(d, axis=1, keepdims=True)
    s = d * inv_mu
    a = s * s
    r = jnp.maximum(1.0 - s, 0.0)
    b = r * r
    wc = 0.5 * (wchat[...] + wchat_t)
    w_full = 0.5 * (wp[...] + wc)
    term = b + (a - b) * w_full
    term = jnp.where(eye, 0.0, term)
    total = jnp.sum(jnp.sum(term, axis=1, keepdims=True), axis=0, keepdims=True)
    loss_ref[...] = total / (_N * (_N - 1))


def _run(sn, tn, ssc, ssr, ttc, ttr, idxc, idxr, interpret=False):
    scr = [pltpu.VMEM((_N, _N), jnp.float32),
           pltpu.VMEM((_N, _N), jnp.int32),
           pltpu.VMEM((_N, _N), jnp.float32),
           pltpu.VMEM((_N, _N), jnp.float32),
           pltpu.VMEM((_N, _N), jnp.float32),
           pltpu.VMEM((_N, _N), jnp.float32)]
    return pl.pallas_call(
        _fused_kernel,
        out_shape=jax.ShapeDtypeStruct((1, 1), jnp.float32),
        scratch_shapes=scr,
        interpret=interpret,
    )(sn, tn, ssc, ssr, ttc, ttr, idxc, idxr)


def kernel(s_emb, t_emb, idx):
    def _norm(x):
        n = jnp.sqrt(jnp.sum(x * x, axis=1, keepdims=True))
        return x / jnp.maximum(n, 1e-12)

    sn = _norm(s_emb)
    tn = _norm(t_emb)
    ss = jnp.sum(sn * sn, axis=1)
    tt = jnp.sum(tn * tn, axis=1)
    idx32 = idx.astype(jnp.int32)
    out = _run(sn, tn,
               ss[:, None], ss[None, :],
               tt[:, None], tt[None, :],
               idx32[:, None], idx32[None, :])
    return out[0, 0]


# speedup vs baseline: 11.5263x; 1.0051x over previous
import jax
import jax.numpy as jnp
from jax import lax
from jax.experimental import pallas as pl
from jax.experimental.pallas import tpu as pltpu

_N = 1024
_K = 10
_H = 5


def _fused_kernel(sn_ref, tn_ref, ssc_ref, ssr_ref, ttc_ref, ttr_ref,
                  idxc_ref, idxr_ref, loss_ref,
                  wp, keys, wnn, ds, mbuf, wchat):
    f32 = jnp.float32
    bf16 = jnp.bfloat16
    i32 = jnp.int32
    iota_row = lax.broadcasted_iota(i32, (_N, _N), 1)
    iota_col = lax.broadcasted_iota(i32, (_N, _N), 0)
    eye = iota_col == iota_row

    tn = tn_ref[...]
    gt = lax.dot_general(tn, tn, (((1,), (1,)), ((), ())),
                         preferred_element_type=f32)
    d2t = jnp.maximum(ttc_ref[...] + ttr_ref[...] - 2.0 * gt, 0.0)
    wp[...] = jnp.exp(-d2t)
    same = idxc_ref[...] == idxr_ref[...]

    bits = lax.bitcast_convert_type(jnp.where(same, 4.0, 4.0 - d2t), i32)
    keys[...] = (bits & i32(~1023)) | (i32(1023) - iota_row)

    sn = sn_ref[...]
    gs = lax.dot_general(sn, sn, (((1,), (1,)), ((), ())),
                         preferred_element_type=f32)
    d2s = ssc_ref[...] + ssr_ref[...] - 2.0 * gs
    ds[...] = jnp.sqrt(jnp.maximum(d2s, 0.0))

    int_min = i32(-2147483648)
    m = jnp.max(keys[...], axis=1, keepdims=True)
    m_half = m
    for k in range(1, _K):
        kv = keys[...]
        m = jnp.max(jnp.where(kv < m, kv, int_min), axis=1, keepdims=True)
        if k == _H - 1:
            m_half = m

    kv = keys[...]
    wnn[...] = jnp.where(kv >= m, 1.0, 0.0)
    gh_bf = jnp.where(kv >= m_half, 1.0 / _H, 0.0).astype(bf16)

    wnn_t = wnn[...].T
    v = jnp.where((wnn[...] > 0) & (wnn_t > 0), 1.0, 0.0)
    denom = jnp.sum(v, axis=1, keepdims=True)
    inv_denom = 1.0 / jnp.where(denom > 0, denom, 1.0)
    v_bf = v.astype(bf16)
    m_mat = lax.dot_general(v_bf, v_bf, (((1,), (1,)), ((), ())),
                            preferred_element_type=f32)
    mbuf[...] = v * m_mat * inv_denom

    wchat[...] = lax.dot_general(gh_bf, mbuf[...].astype(bf16),
                                 (((1,), (0,)), ((), ())),
                                 preferred_element_type=f32)
    wchat_t = wchat[...].T

    d = ds[...]
    inv_mu = float(_N) / jnp.sum(d, axis=1, keepdims=True)
    s = d * inv_mu
    a = s * s
    r = jnp.maximum(1.0 - s, 0.0)
    b = r * r
    wc = 0.5 * (wchat[...] + wchat_t)
    w_full = 0.5 * (wp[...] + wc)
    term = b + (a - b) * w_full
    term = jnp.where(eye, 0.0, term)
    total = jnp.sum(jnp.sum(term, axis=1, keepdims=True), axis=0, keepdims=True)
    loss_ref[...] = total / (_N * (_N - 1))


def _run(sn, tn, ssc, ssr, ttc, ttr, idxc, idxr, interpret=False):
    scr = [pltpu.VMEM((_N, _N), jnp.float32),
           pltpu.VMEM((_N, _N), jnp.int32),
           pltpu.VMEM((_N, _N), jnp.float32),
           pltpu.VMEM((_N, _N), jnp.float32),
           pltpu.VMEM((_N, _N), jnp.float32),
           pltpu.VMEM((_N, _N), jnp.float32)]
    return pl.pallas_call(
        _fused_kernel,
        out_shape=jax.ShapeDtypeStruct((1, 1), jnp.float32),
        scratch_shapes=scr,
        interpret=interpret,
    )(sn, tn, ssc, ssr, ttc, ttr, idxc, idxr)


def kernel(s_emb, t_emb, idx):
    def _norm(x):
        n = jnp.sqrt(jnp.sum(x * x, axis=1, keepdims=True))
        return x / jnp.maximum(n, 1e-12)

    sn = _norm(s_emb)
    tn = _norm(t_emb)
    ss = jnp.sum(sn * sn, axis=1)
    tt = jnp.sum(tn * tn, axis=1)
    idx32 = idx.astype(jnp.int32)
    out = _run(sn, tn,
               ss[:, None], ss[None, :],
               tt[:, None], tt[None, :],
               idx32[:, None], idx32[None, :])
    return out[0, 0]


# speedup vs baseline: 12.8633x; 1.1160x over previous
import jax
import jax.numpy as jnp
from jax import lax
from jax.experimental import pallas as pl
from jax.experimental.pallas import tpu as pltpu

_N = 1024
_K = 10
_H = 5


def _fused_kernel(sn_ref, tn_ref, ssc_ref, ssr_ref, ttc_ref, ttr_ref,
                  idxc_ref, idxr_ref, loss_ref,
                  wp, keys, wnn, ds, mbuf, wchat):
    f32 = jnp.float32
    bf16 = jnp.bfloat16
    i32 = jnp.int32
    iota_row = lax.broadcasted_iota(i32, (_N, _N), 1)
    iota_col = lax.broadcasted_iota(i32, (_N, _N), 0)
    eye = iota_col == iota_row

    tn = tn_ref[...]
    gt = lax.dot_general(tn, tn, (((1,), (1,)), ((), ())),
                         preferred_element_type=f32)
    d2t = jnp.maximum(ttc_ref[...] + ttr_ref[...] - 2.0 * gt, 0.0)
    wp[...] = jnp.exp(-d2t)
    same = idxc_ref[...] == idxr_ref[...]

    bits = lax.bitcast_convert_type(jnp.where(same, 4.0, 4.0 - d2t), i32)
    packed = (bits & i32(~1023)) | (i32(1023) - iota_row)
    keys[...] = lax.bitcast_convert_type(packed, jnp.float32)

    sn = sn_ref[...]
    gs = lax.dot_general(sn, sn, (((1,), (1,)), ((), ())),
                         preferred_element_type=f32)
    d2s = ssc_ref[...] + ssr_ref[...] - 2.0 * gs
    ds[...] = jnp.sqrt(jnp.maximum(d2s, 0.0))

    m = jnp.max(keys[...], axis=1, keepdims=True)
    m_half = m
    for k in range(1, _K):
        kv = keys[...]
        m = jnp.max(jnp.where(kv < m, kv, -jnp.inf), axis=1, keepdims=True)
        if k == _H - 1:
            m_half = m

    kv = keys[...]
    wnn[...] = jnp.where(kv >= m, 1.0, 0.0)
    gh_bf = jnp.where(kv >= m_half, 1.0 / _H, 0.0).astype(bf16)

    wnn_t = wnn[...].T
    v = jnp.where((wnn[...] > 0) & (wnn_t > 0), 1.0, 0.0)
    denom = jnp.sum(v, axis=1, keepdims=True)
    inv_denom = 1.0 / jnp.where(denom > 0, denom, 1.0)
    v_bf = v.astype(bf16)
    m_mat = lax.dot_general(v_bf, v_bf, (((1,), (1,)), ((), ())),
                            preferred_element_type=f32)
    mbuf[...] = v * m_mat * inv_denom

    wchat[...] = lax.dot_general(gh_bf, mbuf[...].astype(bf16),
                                 (((1,), (0,)), ((), ())),
                                 preferred_element_type=f32)
    wchat_t = wchat[...].T

    d = ds[...]
    inv_mu = float(_N) / jnp.sum(d, axis=1, keepdims=True)
    s = d * inv_mu
    a = s * s
    r = jnp.maximum(1.0 - s, 0.0)
    b = r * r
    wc = 0.5 * (wchat[...] + wchat_t)
    w_full = 0.5 * (wp[...] + wc)
    term = b + (a - b) * w_full
    term = jnp.where(eye, 0.0, term)
    total = jnp.sum(jnp.sum(term, axis=1, keepdims=True), axis=0, keepdims=True)
    loss_ref[...] = total / (_N * (_N - 1))


def _run(sn, tn, ssc, ssr, ttc, ttr, idxc, idxr, interpret=False):
    scr = [pltpu.VMEM((_N, _N), jnp.float32)] * 6
    return pl.pallas_call(
        _fused_kernel,
        out_shape=jax.ShapeDtypeStruct((1, 1), jnp.float32),
        scratch_shapes=scr,
        interpret=interpret,
    )(sn, tn, ssc, ssr, ttc, ttr, idxc, idxr)


def kernel(s_emb, t_emb, idx):
    def _norm(x):
        n = jnp.sqrt(jnp.sum(x * x, axis=1, keepdims=True))
        return x / jnp.maximum(n, 1e-12)

    sn = _norm(s_emb)
    tn = _norm(t_emb)
    ss = jnp.sum(sn * sn, axis=1)
    tt = jnp.sum(tn * tn, axis=1)
    idx32 = idx.astype(jnp.int32)
    out = _run(sn, tn,
               ss[:, None], ss[None, :],
               tt[:, None], tt[None, :],
               idx32[:, None], idx32[None, :])
    return out[0, 0]


# speedup vs baseline: 13.3088x; 1.0346x over previous
import jax
import jax.numpy as jnp
from jax import lax
from jax.experimental import pallas as pl
from jax.experimental.pallas import tpu as pltpu

_N = 1024
_K = 10
_H = 5


def _fused_kernel(sn_ref, tn_ref, ssc_ref, ssr_ref, ttc_ref, ttr_ref,
                  idxc_ref, idxr_ref, loss_ref,
                  wp, keys, wnn, ds, mbuf, wchat):
    f32 = jnp.float32
    bf16 = jnp.bfloat16
    i32 = jnp.int32
    iota_row = lax.broadcasted_iota(i32, (_N, _N), 1)
    iota_col = lax.broadcasted_iota(i32, (_N, _N), 0)
    eye = iota_col == iota_row

    tn = tn_ref[...]
    gt = lax.dot_general(tn, tn, (((1,), (1,)), ((), ())),
                         preferred_element_type=f32)
    d2t = jnp.maximum(ttc_ref[...] + ttr_ref[...] - 2.0 * gt, 0.0)
    wp[...] = jnp.exp(-d2t)
    same = idxc_ref[...] == idxr_ref[...]

    bits = lax.bitcast_convert_type(jnp.where(same, 4.0, 4.0 - d2t), i32)
    packed = (bits & i32(~1023)) | (i32(1023) - iota_row)
    keys[...] = lax.bitcast_convert_type(packed, jnp.float32)

    sn = sn_ref[...]
    gs = lax.dot_general(sn, sn, (((1,), (1,)), ((), ())),
                         preferred_element_type=f32)
    d2s = ssc_ref[...] + ssr_ref[...] - 2.0 * gs
    ds[...] = jnp.sqrt(jnp.maximum(d2s, 0.0))

    m = jnp.max(keys[...], axis=1, keepdims=True)
    m_half = m
    for k in range(1, _K):
        kv = keys[...]
        m = jnp.max(jnp.where(kv < m, kv, -jnp.inf), axis=1, keepdims=True)
        if k == _H - 1:
            m_half = m

    kv = keys[...]
    wnn[...] = jnp.where(kv >= m, 1.0, 0.0).astype(bf16)
    gh_bf = jnp.where(kv >= m_half, 1.0 / _H, 0.0).astype(bf16)

    v_bf = wnn[...] * wnn[...].T
    v = v_bf.astype(f32)
    denom = jnp.sum(v, axis=1, keepdims=True)
    inv_denom = 1.0 / jnp.where(denom > 0, denom, 1.0)
    m_mat = lax.dot_general(v_bf, v_bf, (((1,), (1,)), ((), ())),
                            preferred_element_type=f32)
    mbuf[...] = v * m_mat * inv_denom

    wchat[...] = lax.dot_general(gh_bf, mbuf[...].astype(bf16),
                                 (((1,), (0,)), ((), ())),
                                 preferred_element_type=f32)
    wchat_t = wchat[...].T

    d = ds[...]
    inv_mu = float(_N) / jnp.sum(d, axis=1, keepdims=True)
    s = d * inv_mu
    a = s * s
    r = jnp.maximum(1.0 - s, 0.0)
    b = r * r
    wc = 0.5 * (wchat[...] + wchat_t)
    w_full = 0.5 * (wp[...] + wc)
    term = b + (a - b) * w_full
    term = jnp.where(eye, 0.0, term)
    total = jnp.sum(jnp.sum(term, axis=1, keepdims=True), axis=0, keepdims=True)
    loss_ref[...] = total / (_N * (_N - 1))


def _run(sn, tn, ssc, ssr, ttc, ttr, idxc, idxr, interpret=False):
    scr = [pltpu.VMEM((_N, _N), jnp.float32),
           pltpu.VMEM((_N, _N), jnp.float32),
           pltpu.VMEM((_N, _N), jnp.bfloat16),
           pltpu.VMEM((_N, _N), jnp.float32),
           pltpu.VMEM((_N, _N), jnp.float32),
           pltpu.VMEM((_N, _N), jnp.float32)]
    return pl.pallas_call(
        _fused_kernel,
        out_shape=jax.ShapeDtypeStruct((1, 1), jnp.float32),
        scratch_shapes=scr,
        interpret=interpret,
    )(sn, tn, ssc, ssr, ttc, ttr, idxc, idxr)


def kernel(s_emb, t_emb, idx):
    def _norm(x):
        n = jnp.sqrt(jnp.sum(x * x, axis=1, keepdims=True))
        return x / jnp.maximum(n, 1e-12)

    sn = _norm(s_emb)
    tn = _norm(t_emb)
    ss = jnp.sum(sn * sn, axis=1)
    tt = jnp.sum(tn * tn, axis=1)
    idx32 = idx.astype(jnp.int32)
    out = _run(sn, tn,
               ss[:, None], ss[None, :],
               tt[:, None], tt[None, :],
               idx32[:, None], idx32[None, :])
    return out[0, 0]
